# baseline (speedup 1.0000x reference)
import functools

import jax
import jax.numpy as jnp
from jax import lax
from jax.experimental import pallas as pl
from jax.experimental.pallas import tpu as pltpu
from jax.experimental.pallas import tpu_sc as plsc

_N_E = 1024
_E_DIM = 256
_BETA = 0.25
_T = 4608
_BLK = 576
_NBLK = _T // _BLK
_NC = 2
_NS = 16
_ROWS_PER_W = _T // (_NC * _NS)


def _codebook_body(wT_ref, mwT_ref, mb_ref, cbT_ref, cb_ref, sc_ref):
    cbT = (
        jnp.dot(wT_ref[...], mwT_ref[...], preferred_element_type=jnp.float32)
        + mb_ref[...]
    )
    cbT_ref[...] = cbT
    cb_ref[...] = cbT.T
    sc_ref[...] = jnp.sum(cbT ** 2, axis=0, keepdims=True)


def _assign_body(z_ref, cbT_ref, sc_ref, enc_ref, idx_ref, cnt_ref, acc_ref):
    zb = z_ref[0].T
    cbT = cbT_ref[...]
    sz = jnp.sum(zb ** 2, axis=1, keepdims=True)
    sc = sc_ref[...]
    scores = jnp.dot(zb, cbT, preferred_element_type=jnp.float32)
    d = (sz + sc) - 2.0 * scores
    m = jnp.min(d, axis=1, keepdims=True)
    iota = lax.broadcasted_iota(jnp.int32, d.shape, 1)
    idx = jnp.min(jnp.where(d == m, iota, jnp.int32(_N_E)),
                  axis=1, keepdims=True)
    enc = (iota == idx).astype(jnp.float32)
    enc_ref[...] = enc
    idx_ref[...] = idx

    @pl.when(pl.program_id(0) == 0)
    def _():
        acc_ref[...] = jnp.zeros_like(acc_ref)

    acc_ref[...] += jnp.sum(enc, axis=0, keepdims=True)

    @pl.when(pl.program_id(0) == _NBLK - 1)
    def _():
        cnt_ref[...] = acc_ref[...]


def _finalize_body(z_ref, zq_ref, cnt_ref, out_ref, loss_ref, perp_ref,
                   acc_ref):
    z3 = z_ref[0]
    diffT = zq_ref[...].T - z3
    out_ref[0] = z3 + diffT

    @pl.when(pl.program_id(0) == 0)
    def _():
        acc_ref[...] = jnp.zeros_like(acc_ref)

    acc_ref[...] += jnp.full((1, 1), jnp.sum(diffT ** 2), jnp.float32)

    @pl.when(pl.program_id(0) == _NBLK - 1)
    def _():
        m = acc_ref[0, 0] * (1.0 / (_T * _E_DIM))
        loss_ref[...] = jnp.full((1, 1), m + _BETA * m, jnp.float32)
        e_mean = cnt_ref[...] * (1.0 / _T)
        ent = -jnp.sum(e_mean * jnp.log(e_mean + 1e-10))
        perp_ref[...] = jnp.full((1, 1), jnp.exp(ent), jnp.float32)


def _sc_gather(codebook, idx_flat):
    mesh = plsc.VectorSubcoreMesh(core_axis_name="c", subcore_axis_name="s")

    @functools.partial(
        pl.kernel,
        mesh=mesh,
        out_type=jax.ShapeDtypeStruct((_T, _E_DIM), jnp.float32),
        scratch_types=[
            pltpu.VMEM((_ROWS_PER_W,), jnp.int32),
            pltpu.VMEM((_ROWS_PER_W, _E_DIM), jnp.float32),
            pltpu.SemaphoreType.DMA,
        ],
    )
    def k(table_hbm, idx_hbm, out_hbm, idx_v, rows_v, sem):
        wid = lax.axis_index("s") * _NC + lax.axis_index("c")
        base = wid * _ROWS_PER_W
        pltpu.sync_copy(idx_hbm.at[pl.ds(base, _ROWS_PER_W)], idx_v)
        pltpu.async_copy(table_hbm.at[idx_v], rows_v, sem).wait()
        pltpu.sync_copy(rows_v, out_hbm.at[pl.ds(base, _ROWS_PER_W)])

    return k(codebook, idx_flat)


def kernel(z, weights, map_w, map_b):
    f32 = jnp.float32
    wT = weights.T
    mwT = map_w.T
    mb_row = map_b.reshape(1, _N_E)
    z3 = z.reshape(8, _E_DIM, _BLK)

    cbT, codebook, sc_row = pl.pallas_call(
        _codebook_body,
        out_shape=[
            jax.ShapeDtypeStruct((_E_DIM, _N_E), f32),
            jax.ShapeDtypeStruct((_N_E, _E_DIM), f32),
            jax.ShapeDtypeStruct((1, _N_E), f32),
        ],
    )(wT, mwT, mb_row)

    enc, idx_col, counts = pl.pallas_call(
        _assign_body,
        grid=(_NBLK,),
        in_specs=[
            pl.BlockSpec((1, _E_DIM, _BLK), lambda i: (i, 0, 0)),
            pl.BlockSpec((_E_DIM, _N_E), lambda i: (0, 0)),
            pl.BlockSpec((1, _N_E), lambda i: (0, 0)),
        ],
        out_specs=[
            pl.BlockSpec((_BLK, _N_E), lambda i: (i, 0)),
            pl.BlockSpec((_BLK, 1), lambda i: (i, 0)),
            pl.BlockSpec((1, _N_E), lambda i: (0, 0)),
        ],
        out_shape=[
            jax.ShapeDtypeStruct((_T, _N_E), f32),
            jax.ShapeDtypeStruct((_T, 1), jnp.int32),
            jax.ShapeDtypeStruct((1, _N_E), f32),
        ],
        scratch_shapes=[pltpu.VMEM((1, _N_E), f32)],
    )(z3, cbT, sc_row)

    zq_flat = _sc_gather(codebook, idx_col.reshape(_T))

    out3, loss11, perp11 = pl.pallas_call(
        _finalize_body,
        grid=(_NBLK,),
        in_specs=[
            pl.BlockSpec((1, _E_DIM, _BLK), lambda i: (i, 0, 0)),
            pl.BlockSpec((_BLK, _E_DIM), lambda i: (i, 0)),
            pl.BlockSpec((1, _N_E), lambda i: (0, 0)),
        ],
        out_specs=[
            pl.BlockSpec((1, _E_DIM, _BLK), lambda i: (i, 0, 0)),
            pl.BlockSpec((1, 1), lambda i: (0, 0)),
            pl.BlockSpec((1, 1), lambda i: (0, 0)),
        ],
        out_shape=[
            jax.ShapeDtypeStruct((8, _E_DIM, _BLK), f32),
            jax.ShapeDtypeStruct((1, 1), f32),
            jax.ShapeDtypeStruct((1, 1), f32),
        ],
        scratch_shapes=[pltpu.VMEM((1, 1), f32)],
    )(z3, zq_flat, counts)

    z_q = out3.reshape(8, _E_DIM, 24, 24)
    loss = loss11.reshape(())
    perplexity = perp11.reshape(())
    return (loss, z_q, perplexity, enc, idx_col)


# speedup vs baseline: 1.1682x; 1.1682x over previous
import functools

import jax
import jax.numpy as jnp
from jax import lax
from jax.experimental import pallas as pl
from jax.experimental.pallas import tpu as pltpu
from jax.experimental.pallas import tpu_sc as plsc

_N_E = 1024
_E_DIM = 256
_BETA = 0.25
_T = 4608
_BLK = 576
_NBLK = _T // _BLK
_NC = 2
_NS = 16
_ROWS_PER_W = _T // (_NC * _NS)


def _codebook_body(wT_ref, mwT_ref, mb_ref, cbT_ref, cb_ref, sc_ref):
    cbT = (
        jnp.dot(wT_ref[...], mwT_ref[...], preferred_element_type=jnp.float32)
        + mb_ref[...]
    )
    cbT_ref[...] = cbT
    cb_ref[...] = cbT.T
    sc_ref[...] = jnp.sum(cbT ** 2, axis=0, keepdims=True)


def _assign_body(z_ref, cbT_ref, sc_ref, enc_ref, idx_ref, cnt_ref, acc_ref):
    zb = z_ref[...]
    cbT = cbT_ref[...]
    sz = jnp.sum(zb ** 2, axis=1, keepdims=True)
    sc = sc_ref[...]
    scores = jnp.dot(zb, cbT, preferred_element_type=jnp.float32)
    d = (sz + sc) - 2.0 * scores
    m = jnp.min(d, axis=1, keepdims=True)
    iota = lax.broadcasted_iota(jnp.int32, d.shape, 1)
    idx = jnp.min(jnp.where(d == m, iota, jnp.int32(_N_E)),
                  axis=1, keepdims=True)
    enc = (iota == idx).astype(jnp.float32)
    enc_ref[...] = enc
    idx_ref[...] = idx

    @pl.when(pl.program_id(0) == 0)
    def _():
        acc_ref[...] = jnp.zeros_like(acc_ref)

    acc_ref[...] += jnp.sum(enc, axis=0, keepdims=True)

    @pl.when(pl.program_id(0) == _NBLK - 1)
    def _():
        cnt_ref[...] = acc_ref[...]


def _finalize_body(zp_ref, zq_ref, cnt_ref, out_ref, loss_ref, perp_ref):
    zp = zp_ref[...]
    diff = zq_ref[...] - zp
    out_ref[...] = zp + diff
    m = jnp.mean(diff ** 2)
    loss_ref[...] = jnp.full((1, 1), m + _BETA * m, jnp.float32)
    e_mean = cnt_ref[...] * (1.0 / _T)
    ent = -jnp.sum(e_mean * jnp.log(e_mean + 1e-10))
    perp_ref[...] = jnp.full((1, 1), jnp.exp(ent), jnp.float32)


def _sc_gather(codebook, idx_flat):
    mesh = plsc.VectorSubcoreMesh(core_axis_name="c", subcore_axis_name="s")

    @functools.partial(
        pl.kernel,
        mesh=mesh,
        out_type=jax.ShapeDtypeStruct((_T, _E_DIM), jnp.float32),
        scratch_types=[
            pltpu.VMEM((_ROWS_PER_W,), jnp.int32),
            pltpu.VMEM((_ROWS_PER_W, _E_DIM), jnp.float32),
            pltpu.SemaphoreType.DMA,
        ],
    )
    def k(table_hbm, idx_hbm, out_hbm, idx_v, rows_v, sem):
        wid = lax.axis_index("s") * _NC + lax.axis_index("c")
        base = wid * _ROWS_PER_W
        pltpu.sync_copy(idx_hbm.at[pl.ds(base, _ROWS_PER_W)], idx_v)
        pltpu.async_copy(table_hbm.at[idx_v], rows_v, sem).wait()
        pltpu.sync_copy(rows_v, out_hbm.at[pl.ds(base, _ROWS_PER_W)])

    return k(codebook, idx_flat)


def kernel(z, weights, map_w, map_b):
    f32 = jnp.float32
    wT = weights.T
    mwT = map_w.T
    mb_row = map_b.reshape(1, _N_E)
    zp = jnp.transpose(z, (0, 2, 3, 1))
    z_flat = zp.reshape(_T, _E_DIM)

    cbT, codebook, sc_row = pl.pallas_call(
        _codebook_body,
        out_shape=[
            jax.ShapeDtypeStruct((_E_DIM, _N_E), f32),
            jax.ShapeDtypeStruct((_N_E, _E_DIM), f32),
            jax.ShapeDtypeStruct((1, _N_E), f32),
        ],
    )(wT, mwT, mb_row)

    enc, idx_col, counts = pl.pallas_call(
        _assign_body,
        grid=(_NBLK,),
        in_specs=[
            pl.BlockSpec((_BLK, _E_DIM), lambda i: (i, 0)),
            pl.BlockSpec((_E_DIM, _N_E), lambda i: (0, 0)),
            pl.BlockSpec((1, _N_E), lambda i: (0, 0)),
        ],
        out_specs=[
            pl.BlockSpec((_BLK, _N_E), lambda i: (i, 0)),
            pl.BlockSpec((_BLK, 1), lambda i: (i, 0)),
            pl.BlockSpec((1, _N_E), lambda i: (0, 0)),
        ],
        out_shape=[
            jax.ShapeDtypeStruct((_T, _N_E), f32),
            jax.ShapeDtypeStruct((_T, 1), jnp.int32),
            jax.ShapeDtypeStruct((1, _N_E), f32),
        ],
        scratch_shapes=[pltpu.VMEM((1, _N_E), f32)],
    )(z_flat, cbT, sc_row)

    zq_flat = _sc_gather(codebook, idx_col.reshape(_T))

    zq_fwd, loss11, perp11 = pl.pallas_call(
        _finalize_body,
        out_shape=[
            jax.ShapeDtypeStruct((_T, _E_DIM), f32),
            jax.ShapeDtypeStruct((1, 1), f32),
            jax.ShapeDtypeStruct((1, 1), f32),
        ],
    )(z_flat, zq_flat, counts)

    z_q = jnp.transpose(zq_fwd.reshape(8, 24, 24, _E_DIM), (0, 3, 1, 2))
    loss = loss11.reshape(())
    perplexity = perp11.reshape(())
    return (loss, z_q, perplexity, enc, idx_col)


# speedup vs baseline: 1.4229x; 1.2181x over previous
import functools

import jax
import jax.numpy as jnp
from jax import lax
from jax.experimental import pallas as pl
from jax.experimental.pallas import tpu as pltpu
from jax.experimental.pallas import tpu_sc as plsc

_N_E = 1024
_E_DIM = 256
_BETA = 0.25
_T = 4608
_BLK = 576
_NBLK = _T // _BLK
_NC = 2
_NS = 16
_ROWS_PER_W = _T // (_NC * _NS)


def _codebook_body(w_ref, mw_ref, mb_ref, cbT_ref, cb_ref, sc_ref):
    cbT = (
        lax.dot_general(w_ref[...], mw_ref[...], (((0,), (1,)), ((), ())),
                        preferred_element_type=jnp.float32)
        + mb_ref[...]
    )
    cbT_ref[...] = cbT
    cb_ref[...] = cbT.T
    sc_ref[...] = jnp.sum(cbT ** 2, axis=0, keepdims=True)


def _assign_body(z_ref, cbT_ref, sc_ref, enc_ref, idx_ref, cnt_ref, acc_ref):
    zb = z_ref[...]
    cbT = cbT_ref[...]
    sz = jnp.sum(zb ** 2, axis=1, keepdims=True)
    sc = sc_ref[...]
    scores = jnp.dot(zb, cbT, preferred_element_type=jnp.float32)
    d = (sz + sc) - 2.0 * scores
    m = jnp.min(d, axis=1, keepdims=True)
    iota = lax.broadcasted_iota(jnp.int32, d.shape, 1)
    idx = jnp.min(jnp.where(d == m, iota, jnp.int32(_N_E)),
                  axis=1, keepdims=True)
    enc = (iota == idx).astype(jnp.float32)
    enc_ref[...] = enc
    idx_ref[...] = idx

    @pl.when(pl.program_id(0) == 0)
    def _():
        acc_ref[...] = jnp.zeros_like(acc_ref)

    acc_ref[...] += jnp.sum(enc, axis=0, keepdims=True)

    @pl.when(pl.program_id(0) == _NBLK - 1)
    def _():
        cnt_ref[...] = acc_ref[...]


def _finalize_body(zp_ref, zq_ref, cnt_ref, out_ref, loss_ref, perp_ref):
    zp = zp_ref[...]
    diff = zq_ref[...] - zp
    out_ref[...] = zp + diff
    m = jnp.mean(diff ** 2)
    loss_ref[...] = jnp.full((1, 1), m + _BETA * m, jnp.float32)
    e_mean = cnt_ref[...] * (1.0 / _T)
    ent = -jnp.sum(e_mean * jnp.log(e_mean + 1e-10))
    perp_ref[...] = jnp.full((1, 1), jnp.exp(ent), jnp.float32)


def _sc_gather(codebook, idx_flat):
    mesh = plsc.VectorSubcoreMesh(core_axis_name="c", subcore_axis_name="s")

    @functools.partial(
        pl.kernel,
        mesh=mesh,
        out_type=jax.ShapeDtypeStruct((_T, _E_DIM), jnp.float32),
        scratch_types=[
            pltpu.VMEM((_ROWS_PER_W,), jnp.int32),
            pltpu.VMEM((_ROWS_PER_W, _E_DIM), jnp.float32),
            pltpu.SemaphoreType.DMA,
        ],
    )
    def k(table_hbm, idx_hbm, out_hbm, idx_v, rows_v, sem):
        wid = lax.axis_index("s") * _NC + lax.axis_index("c")
        base = wid * _ROWS_PER_W
        pltpu.sync_copy(idx_hbm.at[pl.ds(base, _ROWS_PER_W)], idx_v)
        pltpu.async_copy(table_hbm.at[idx_v], rows_v, sem).wait()
        pltpu.sync_copy(rows_v, out_hbm.at[pl.ds(base, _ROWS_PER_W)])

    return k(codebook, idx_flat)


def kernel(z, weights, map_w, map_b):
    f32 = jnp.float32
    mb_row = map_b.reshape(1, _N_E)
    zp = jnp.transpose(z, (0, 2, 3, 1))
    z_flat = zp.reshape(_T, _E_DIM)

    cbT, codebook, sc_row = pl.pallas_call(
        _codebook_body,
        out_shape=[
            jax.ShapeDtypeStruct((_E_DIM, _N_E), f32),
            jax.ShapeDtypeStruct((_N_E, _E_DIM), f32),
            jax.ShapeDtypeStruct((1, _N_E), f32),
        ],
    )(weights, map_w, mb_row)

    enc, idx_col, counts = pl.pallas_call(
        _assign_body,
        grid=(_NBLK,),
        in_specs=[
            pl.BlockSpec((_BLK, _E_DIM), lambda i: (i, 0)),
            pl.BlockSpec((_E_DIM, _N_E), lambda i: (0, 0)),
            pl.BlockSpec((1, _N_E), lambda i: (0, 0)),
        ],
        out_specs=[
            pl.BlockSpec((_BLK, _N_E), lambda i: (i, 0)),
            pl.BlockSpec((_BLK, 1), lambda i: (i, 0)),
            pl.BlockSpec((1, _N_E), lambda i: (0, 0)),
        ],
        out_shape=[
            jax.ShapeDtypeStruct((_T, _N_E), f32),
            jax.ShapeDtypeStruct((_T, 1), jnp.int32),
            jax.ShapeDtypeStruct((1, _N_E), f32),
        ],
        scratch_shapes=[pltpu.VMEM((1, _N_E), f32)],
    )(z_flat, cbT, sc_row)

    zq_flat = _sc_gather(codebook, idx_col.reshape(_T))

    zq_fwd, loss11, perp11 = pl.pallas_call(
        _finalize_body,
        out_shape=[
            jax.ShapeDtypeStruct((_T, _E_DIM), f32),
            jax.ShapeDtypeStruct((1, 1), f32),
            jax.ShapeDtypeStruct((1, 1), f32),
        ],
    )(z_flat, zq_flat, counts)

    z_q = jnp.transpose(zq_fwd.reshape(8, 24, 24, _E_DIM), (0, 3, 1, 2))
    loss = loss11.reshape(())
    perplexity = perp11.reshape(())
    return (loss, z_q, perplexity, enc, idx_col)
